# baseline (speedup 1.0000x reference)
import functools

import jax
import jax.numpy as jnp
from jax import lax
from jax.experimental import pallas as pl
from jax.experimental.pallas import tpu as pltpu
from jax.experimental.pallas import tpu_sc as plsc

N_NODES = 10000
HIDDEN = 128
N_EDGES = 320000
NUM_REL = 64

NUM_CORES = 2
NUM_SUBCORES = 16
NW = NUM_CORES * NUM_SUBCORES
EDGES_PER_W = N_EDGES // NW
CHUNK = 80
NCHUNK = EDGES_PER_W // CHUNK
N_SBLK = N_NODES // CHUNK
REL_PAD = 128
ZFLAT = 8000
CNT_WORDS = N_NODES * REL_PAD
CNT_PER_TILE = CNT_WORDS // NUM_SUBCORES


def _rows_body(ns_hbm, ei_hbm, out_s,
               rows0, rows1, rows2, sb0, sb1, sb2, db0, db1, db2, s_sh,
               g0, g1, g2, l0, l1, l2, m0, m1, m2):
    c = lax.axis_index("c")
    s = lax.axis_index("s")
    w = c * NUM_SUBCORES + s
    base_s = w * EDGES_PER_W
    base_d = N_EDGES + w * EDGES_PER_W
    rows = (rows0, rows1, rows2)
    srcb = (sb0, sb1, sb2)
    dstb = (db0, db1, db2)
    gsem = (g0, g1, g2)
    lsem = (l0, l1, l2)
    msem = (m0, m1, m2)

    def zrow_body(i, carry):
        for j in range(HIDDEN // 16):
            rows0[i, pl.ds(j * 16, 16)] = jnp.zeros((16,), jnp.float32)
        return carry
    lax.fori_loop(0, CHUNK, zrow_body, 0)

    zcps = []
    for k in range(N_SBLK // NUM_SUBCORES):
        blk = s + NUM_SUBCORES * k
        zcps.append(pltpu.async_copy(rows0, s_sh.at[pl.ds(blk * CHUNK, CHUNK)], g0))
    blk_raw = s + NUM_SUBCORES * (N_SBLK // NUM_SUBCORES)
    blk = jnp.minimum(blk_raw, N_SBLK - 1)
    @pl.when(blk_raw < N_SBLK)
    def _():
        pltpu.async_copy(rows0, s_sh.at[pl.ds(blk * CHUNK, CHUNK)], g0).wait()
    for cp in zcps:
        cp.wait()

    plsc.subcore_barrier()

    def load_idx(i, b):
        pltpu.async_copy(
            ei_hbm.at[pl.ds(base_s + i * CHUNK, CHUNK)], srcb[b], lsem[b])
        pltpu.async_copy(
            ei_hbm.at[pl.ds(base_d + i * CHUNK, CHUNK)], dstb[b], msem[b])

    def wait_idx(b):
        pltpu.make_async_copy(
            ei_hbm.at[pl.ds(0, CHUNK)], srcb[b], lsem[b]).wait()
        pltpu.make_async_copy(
            ei_hbm.at[pl.ds(0, CHUNK)], dstb[b], msem[b]).wait()

    def fire(b):
        return pltpu.async_copy(ns_hbm.at[srcb[b]], rows[b], gsem[b])

    load_idx(0, 0)
    wait_idx(0)
    fire(0)
    load_idx(1, 1)
    wait_idx(1)
    fire(1)
    load_idx(2, 2)

    def tri_body(k, carry):
        for j in range(3):
            i = 3 * k + j
            b = j
            b2 = (j + 2) % 3
            pltpu.make_async_copy(ns_hbm.at[srcb[b]], rows[b], gsem[b]).wait()
            wait_idx(b2)
            fire(b2)
            pltpu.sync_copy(rows[b], s_sh.at[dstb[b]], add=True)
            @pl.when(i + 3 < NCHUNK)
            def _():
                load_idx(i + 3, b)
        return carry
    lax.fori_loop(0, (NCHUNK - 2) // 3, tri_body, 0)

    for i in (NCHUNK - 2, NCHUNK - 1):
        b = i % 3
        pltpu.make_async_copy(ns_hbm.at[srcb[b]], rows[b], gsem[b]).wait()
        pltpu.sync_copy(rows[b], s_sh.at[dstb[b]], add=True)

    plsc.subcore_barrier()

    ocps = [None, None, None]
    for k in range(N_SBLK // NUM_SUBCORES):
        b = k % 3
        blk = s + NUM_SUBCORES * k
        r0 = blk * CHUNK
        if ocps[b] is not None:
            ocps[b].wait()
        pltpu.sync_copy(s_sh.at[pl.ds(r0, CHUNK)], rows[b])
        ocps[b] = pltpu.async_copy(rows[b], out_s.at[c, pl.ds(r0, CHUNK)], gsem[b])
    ocps[2].wait()
    ocps[2] = None
    blk_raw = s + NUM_SUBCORES * (N_SBLK // NUM_SUBCORES)
    blk = jnp.minimum(blk_raw, N_SBLK - 1)
    @pl.when(blk_raw < N_SBLK)
    def _():
        r0 = blk * CHUNK
        pltpu.sync_copy(s_sh.at[pl.ds(r0, CHUNK)], rows2)
        pltpu.async_copy(rows2, out_s.at[c, pl.ds(r0, CHUNK)], g2).wait()
    for b in range(3):
        if ocps[b] is not None:
            ocps[b].wait()


_sc_rows = functools.partial(
    pl.kernel,
    out_type=jax.ShapeDtypeStruct((NUM_CORES, N_NODES, HIDDEN), jnp.float32),
    mesh=plsc.VectorSubcoreMesh(core_axis_name="c", subcore_axis_name="s"),
    scratch_types=[
        pltpu.VMEM((CHUNK, HIDDEN), jnp.float32),
        pltpu.VMEM((CHUNK, HIDDEN), jnp.float32),
        pltpu.VMEM((CHUNK, HIDDEN), jnp.float32),
        pltpu.VMEM((CHUNK,), jnp.int32),
        pltpu.VMEM((CHUNK,), jnp.int32),
        pltpu.VMEM((CHUNK,), jnp.int32),
        pltpu.VMEM((CHUNK,), jnp.int32),
        pltpu.VMEM((CHUNK,), jnp.int32),
        pltpu.VMEM((CHUNK,), jnp.int32),
        pltpu.VMEM_SHARED((N_NODES, HIDDEN), jnp.float32),
        pltpu.SemaphoreType.DMA,
        pltpu.SemaphoreType.DMA,
        pltpu.SemaphoreType.DMA,
        pltpu.SemaphoreType.DMA,
        pltpu.SemaphoreType.DMA,
        pltpu.SemaphoreType.DMA,
        pltpu.SemaphoreType.DMA,
        pltpu.SemaphoreType.DMA,
        pltpu.SemaphoreType.DMA,
    ],
)(_rows_body)


def _cnt_body(ei_hbm, rel_hbm, out_c,
              db0, db1, db2, rb0, rb1, rb2, cidx_all, ones_v, zflat,
              cnt_sh, ssem, l0, l1, l2, m0, m1, m2):
    c = lax.axis_index("c")
    s = lax.axis_index("s")
    w = c * NUM_SUBCORES + s
    base_d = N_EDGES + w * EDGES_PER_W
    base_r = w * EDGES_PER_W
    dstb = (db0, db1, db2)
    relb = (rb0, rb1, rb2)
    lsem = (l0, l1, l2)
    msem = (m0, m1, m2)

    def zflat_body(i, carry):
        zflat[pl.ds(i * 16, 16)] = jnp.zeros((16,), jnp.float32)
        return carry
    lax.fori_loop(0, ZFLAT // 16, zflat_body, 0)
    for j in range(CHUNK // 16):
        ones_v[pl.ds(j * 16, 16)] = jnp.ones((16,), jnp.float32)

    zcps = []
    for k in range(CNT_PER_TILE // ZFLAT):
        zcps.append(pltpu.async_copy(
            zflat, cnt_sh.at[pl.ds(s * CNT_PER_TILE + k * ZFLAT, ZFLAT)], ssem))
    for cp in zcps:
        cp.wait()

    plsc.subcore_barrier()

    def load_idx(i, b):
        pltpu.async_copy(
            ei_hbm.at[pl.ds(base_d + i * CHUNK, CHUNK)], dstb[b], lsem[b])
        pltpu.async_copy(
            rel_hbm.at[pl.ds(base_r + i * CHUNK, CHUNK)], relb[b], msem[b])

    def wait_idx(b):
        pltpu.make_async_copy(
            ei_hbm.at[pl.ds(0, CHUNK)], dstb[b], lsem[b]).wait()
        pltpu.make_async_copy(
            rel_hbm.at[pl.ds(0, CHUNK)], relb[b], msem[b]).wait()

    load_idx(0, 0)
    load_idx(1, 1)
    load_idx(2, 2)

    def tri_body(k, carry):
        for j in range(3):
            i = 3 * k + j
            b = j
            wait_idx(b)
            for u in range(CHUNK // 16):
                d16 = dstb[b][pl.ds(u * 16, 16)]
                r16 = relb[b][pl.ds(u * 16, 16)]
                r16 = jnp.minimum(jnp.maximum(r16, 0), NUM_REL - 1)
                cidx_all[i, pl.ds(u * 16, 16)] = d16 * REL_PAD + r16
            @pl.when(i + 3 < NCHUNK)
            def _():
                load_idx(i + 3, b)
            pltpu.async_copy(ones_v, cnt_sh.at[cidx_all.at[i]], ssem, add=True)
        return carry
    lax.fori_loop(0, (NCHUNK - 2) // 3, tri_body, 0)

    for i in (NCHUNK - 2, NCHUNK - 1):
        b = i % 3
        wait_idx(b)
        for u in range(CHUNK // 16):
            d16 = dstb[b][pl.ds(u * 16, 16)]
            r16 = relb[b][pl.ds(u * 16, 16)]
            r16 = jnp.minimum(jnp.maximum(r16, 0), NUM_REL - 1)
            cidx_all[i, pl.ds(u * 16, 16)] = d16 * REL_PAD + r16
        pltpu.async_copy(ones_v, cnt_sh.at[cidx_all.at[i]], ssem, add=True)

    def drain_body(i, carry):
        pltpu.make_async_copy(ones_v, cnt_sh.at[pl.ds(0, CHUNK)], ssem).wait()
        return carry
    lax.fori_loop(0, NCHUNK, drain_body, 0)

    plsc.subcore_barrier()

    for k in range(CNT_PER_TILE // ZFLAT):
        o0 = s * CNT_PER_TILE + k * ZFLAT
        pltpu.sync_copy(cnt_sh.at[pl.ds(o0, ZFLAT)], zflat)
        pltpu.sync_copy(zflat, out_c.at[pl.ds(c * CNT_WORDS + o0, ZFLAT)])


_sc_counts = functools.partial(
    pl.kernel,
    out_type=jax.ShapeDtypeStruct((NUM_CORES * CNT_WORDS,), jnp.float32),
    mesh=plsc.VectorSubcoreMesh(core_axis_name="c", subcore_axis_name="s"),
    scratch_types=[
        pltpu.VMEM((CHUNK,), jnp.int32),
        pltpu.VMEM((CHUNK,), jnp.int32),
        pltpu.VMEM((CHUNK,), jnp.int32),
        pltpu.VMEM((CHUNK,), jnp.int32),
        pltpu.VMEM((CHUNK,), jnp.int32),
        pltpu.VMEM((CHUNK,), jnp.int32),
        pltpu.VMEM((NCHUNK, CHUNK), jnp.int32),
        pltpu.VMEM((CHUNK,), jnp.float32),
        pltpu.VMEM((ZFLAT,), jnp.float32),
        pltpu.VMEM_SHARED((CNT_WORDS,), jnp.float32),
        pltpu.SemaphoreType.DMA,
        pltpu.SemaphoreType.DMA,
        pltpu.SemaphoreType.DMA,
        pltpu.SemaphoreType.DMA,
        pltpu.SemaphoreType.DMA,
        pltpu.SemaphoreType.DMA,
        pltpu.SemaphoreType.DMA,
    ],
)(_cnt_body)


BLOCK_ROWS = 1000


def _tc_body(ns_ref, s2_ref, c2_ref, rel_ref, self_w_ref, self_b_ref,
             msg_w_ref, msg_b_ref, out_ref):
    s_tot = s2_ref[0] + s2_ref[1]
    cm = c2_ref[0] + c2_ref[1]
    deg = jnp.sum(cm, axis=1, keepdims=True)
    rel_sum = lax.dot_general(cm, rel_ref[...], (((1,), (0,)), ((), ())),
                              preferred_element_type=jnp.float32)
    numer = lax.dot_general(s_tot + rel_sum, msg_w_ref[...],
                            (((1,), (1,)), ((), ())),
                            preferred_element_type=jnp.float32)
    numer = numer + deg * msg_b_ref[...]
    agg = numer / jnp.maximum(deg, 1.0)
    out_ref[...] = lax.dot_general(ns_ref[...], self_w_ref[...],
                                   (((1,), (1,)), ((), ())),
                                   preferred_element_type=jnp.float32) \
        + self_b_ref[...] + agg


def _tc_combine(ns, s2, c2, rel_emb, self_w, self_b, msg_w, msg_b):
    grid = (N_NODES // BLOCK_ROWS,)
    return pl.pallas_call(
        _tc_body,
        grid=grid,
        in_specs=[
            pl.BlockSpec((BLOCK_ROWS, HIDDEN), lambda i: (i, 0)),
            pl.BlockSpec((NUM_CORES, BLOCK_ROWS, HIDDEN), lambda i: (0, i, 0)),
            pl.BlockSpec((NUM_CORES, BLOCK_ROWS, REL_PAD), lambda i: (0, i, 0)),
            pl.BlockSpec((REL_PAD, HIDDEN), lambda i: (0, 0)),
            pl.BlockSpec((HIDDEN, HIDDEN), lambda i: (0, 0)),
            pl.BlockSpec((1, HIDDEN), lambda i: (0, 0)),
            pl.BlockSpec((HIDDEN, HIDDEN), lambda i: (0, 0)),
            pl.BlockSpec((1, HIDDEN), lambda i: (0, 0)),
        ],
        out_specs=pl.BlockSpec((BLOCK_ROWS, HIDDEN), lambda i: (i, 0)),
        out_shape=jax.ShapeDtypeStruct((N_NODES, HIDDEN), jnp.float32),
    )(ns, s2, c2, rel_emb, self_w, self_b, msg_w, msg_b)


def kernel(node_states, edge_index, edge_type_ids, self_W, self_b, msg_W, msg_b, rel_emb):
    ei_flat = edge_index.reshape(2 * N_EDGES)
    s2 = _sc_rows(node_states, ei_flat)
    c2 = _sc_counts(ei_flat, edge_type_ids)
    c2 = c2.reshape(NUM_CORES, N_NODES, REL_PAD)
    rel_pad = jnp.concatenate(
        [rel_emb, jnp.zeros((REL_PAD - NUM_REL, HIDDEN), jnp.float32)], axis=0)
    return _tc_combine(node_states, s2, c2, rel_pad, self_W,
                       self_b.reshape(1, HIDDEN), msg_W, msg_b.reshape(1, HIDDEN))


# speedup vs baseline: 1.2955x; 1.2955x over previous
import functools

import jax
import jax.numpy as jnp
from jax import lax
from jax.experimental import pallas as pl
from jax.experimental.pallas import tpu as pltpu
from jax.experimental.pallas import tpu_sc as plsc

N_NODES = 10000
HIDDEN = 128
N_EDGES = 320000
NUM_REL = 64

NUM_CORES = 2
NUM_SUBCORES = 16
NW = NUM_CORES * NUM_SUBCORES
EDGES_PER_W = N_EDGES // NW
CHUNK = 80
NCHUNK = EDGES_PER_W // CHUNK
N_SBLK = N_NODES // CHUNK
REL_PAD = NUM_REL
ZFLAT = 8000
CNT_WORDS = N_NODES * REL_PAD
CNT_PER_TILE = CNT_WORDS // NUM_SUBCORES


def _rows_body(ns_hbm, src_hbm, dst_hbm, out_s,
               rows0, rows1, rows2, sb0, sb1, sb2, dst_all, s_sh,
               g0, g1, g2, l0, l1, l2, p0):
    c = lax.axis_index("c")
    s = lax.axis_index("s")
    w = c * NUM_SUBCORES + s
    base = w * EDGES_PER_W
    rows = (rows0, rows1, rows2)
    srcb = (sb0, sb1, sb2)
    gsem = (g0, g1, g2)
    lsem = (l0, l1, l2)

    def dfire_body(i, carry):
        pltpu.async_copy(dst_hbm.at[pl.ds(base + i * CHUNK, CHUNK)],
                         dst_all.at[i], p0)
        return carry
    lax.fori_loop(0, NCHUNK, dfire_body, 0)

    def zrow_body(i, carry):
        for j in range(HIDDEN // 16):
            rows0[i, pl.ds(j * 16, 16)] = jnp.zeros((16,), jnp.float32)
        return carry
    lax.fori_loop(0, CHUNK, zrow_body, 0)

    zcps = []
    for k in range(N_SBLK // NUM_SUBCORES):
        blk = s + NUM_SUBCORES * k
        zcps.append(pltpu.async_copy(rows0, s_sh.at[pl.ds(blk * CHUNK, CHUNK)], g0))
    blk_raw = s + NUM_SUBCORES * (N_SBLK // NUM_SUBCORES)
    blk = jnp.minimum(blk_raw, N_SBLK - 1)
    @pl.when(blk_raw < N_SBLK)
    def _():
        pltpu.async_copy(rows0, s_sh.at[pl.ds(blk * CHUNK, CHUNK)], g0).wait()
    for cp in zcps:
        cp.wait()

    def ddrain_body(i, carry):
        pltpu.make_async_copy(dst_hbm.at[pl.ds(0, CHUNK)], dst_all.at[0], p0).wait()
        return carry
    lax.fori_loop(0, NCHUNK, ddrain_body, 0)

    plsc.subcore_barrier()

    def load_src(i, b):
        return pltpu.async_copy(
            src_hbm.at[pl.ds(base + i * CHUNK, CHUNK)], srcb[b], lsem[b])

    def fire(b):
        return pltpu.async_copy(ns_hbm.at[srcb[b]], rows[b], gsem[b])

    load_src(0, 0).wait()
    fire(0)
    load_src(1, 1).wait()
    fire(1)
    load_src(2, 2)

    def tri_body(k, carry):
        for j in range(3):
            i = 3 * k + j
            b = j
            b2 = (j + 2) % 3
            pltpu.make_async_copy(ns_hbm.at[srcb[b]], rows[b], gsem[b]).wait()
            @pl.when(i + 3 < NCHUNK)
            def _():
                load_src(i + 3, b)
            pltpu.make_async_copy(
                src_hbm.at[pl.ds(0, CHUNK)], srcb[b2], lsem[b2]).wait()
            fire(b2)
            pltpu.sync_copy(rows[b], s_sh.at[dst_all.at[i]], add=True)
        return carry
    lax.fori_loop(0, (NCHUNK - 2) // 3, tri_body, 0)

    for i in (NCHUNK - 2, NCHUNK - 1):
        b = i % 3
        pltpu.make_async_copy(ns_hbm.at[srcb[b]], rows[b], gsem[b]).wait()
        pltpu.sync_copy(rows[b], s_sh.at[dst_all.at[i]], add=True)

    plsc.subcore_barrier()

    rcps = []
    for k in range(N_SBLK // NUM_SUBCORES):
        blk = s + NUM_SUBCORES * k
        r0 = blk * CHUNK
        rcps.append(pltpu.async_copy(
            s_sh.at[pl.ds(r0, CHUNK)], out_s.at[c, pl.ds(r0, CHUNK)], p0))
    blk_raw = s + NUM_SUBCORES * (N_SBLK // NUM_SUBCORES)
    blk = jnp.minimum(blk_raw, N_SBLK - 1)
    @pl.when(blk_raw < N_SBLK)
    def _():
        r0 = blk * CHUNK
        pltpu.async_copy(
            s_sh.at[pl.ds(r0, CHUNK)], out_s.at[c, pl.ds(r0, CHUNK)], p0).wait()
    for cp in rcps:
        cp.wait()


_sc_rows = functools.partial(
    pl.kernel,
    out_type=jax.ShapeDtypeStruct((NUM_CORES, N_NODES, HIDDEN), jnp.float32),
    mesh=plsc.VectorSubcoreMesh(core_axis_name="c", subcore_axis_name="s"),
    scratch_types=[
        pltpu.VMEM((CHUNK, HIDDEN), jnp.float32),
        pltpu.VMEM((CHUNK, HIDDEN), jnp.float32),
        pltpu.VMEM((CHUNK, HIDDEN), jnp.float32),
        pltpu.VMEM((CHUNK,), jnp.int32),
        pltpu.VMEM((CHUNK,), jnp.int32),
        pltpu.VMEM((CHUNK,), jnp.int32),
        pltpu.VMEM((NCHUNK, CHUNK), jnp.int32),
        pltpu.VMEM_SHARED((N_NODES, HIDDEN), jnp.float32),
        pltpu.SemaphoreType.DMA,
        pltpu.SemaphoreType.DMA,
        pltpu.SemaphoreType.DMA,
        pltpu.SemaphoreType.DMA,
        pltpu.SemaphoreType.DMA,
        pltpu.SemaphoreType.DMA,
        pltpu.SemaphoreType.DMA,
    ],
)(_rows_body)


def _cnt_body(dst_hbm, rel_hbm, out_c,
              dst_v, rel_v, cidx_all, ones_v, zflat, cnt_sh, ssem, p0, p1):
    c = lax.axis_index("c")
    s = lax.axis_index("s")
    w = c * NUM_SUBCORES + s
    base = w * EDGES_PER_W

    for j in range(CHUNK // 16):
        ones_v[pl.ds(j * 16, 16)] = jnp.ones((16,), jnp.float32)

    def zflat_body(i, carry):
        zflat[pl.ds(i * 16, 16)] = jnp.zeros((16,), jnp.float32)
        return carry
    lax.fori_loop(0, ZFLAT // 16, zflat_body, 0)

    zcps = []
    for k in range(CNT_PER_TILE // ZFLAT):
        zcps.append(pltpu.async_copy(
            zflat, cnt_sh.at[pl.ds(s * CNT_PER_TILE + k * ZFLAT, ZFLAT)], ssem))

    def pfire_body(i, carry):
        pltpu.async_copy(dst_hbm.at[pl.ds(base + i * CHUNK, CHUNK)],
                         dst_v.at[i], p0)
        pltpu.async_copy(rel_hbm.at[pl.ds(base + i * CHUNK, CHUNK)],
                         rel_v.at[i], p1)
        return carry
    lax.fori_loop(0, NCHUNK, pfire_body, 0)

    def pdrain_body(i, carry):
        pltpu.make_async_copy(dst_hbm.at[pl.ds(0, CHUNK)], dst_v.at[0], p0).wait()
        pltpu.make_async_copy(rel_hbm.at[pl.ds(0, CHUNK)], rel_v.at[0], p1).wait()
        return carry
    lax.fori_loop(0, NCHUNK, pdrain_body, 0)
    for cp in zcps:
        cp.wait()

    plsc.subcore_barrier()

    def chunk_body(i, carry):
        for u in range(CHUNK // 16):
            d16 = dst_v[i, pl.ds(u * 16, 16)]
            r16 = rel_v[i, pl.ds(u * 16, 16)]
            r16 = jnp.minimum(jnp.maximum(r16, 0), NUM_REL - 1)
            cidx_all[i, pl.ds(u * 16, 16)] = d16 * REL_PAD + r16
        pltpu.async_copy(ones_v, cnt_sh.at[cidx_all.at[i]], ssem, add=True)
        return carry
    lax.fori_loop(0, NCHUNK, chunk_body, 0)

    def drain_body(i, carry):
        pltpu.make_async_copy(ones_v, cnt_sh.at[pl.ds(0, CHUNK)], ssem).wait()
        return carry
    lax.fori_loop(0, NCHUNK, drain_body, 0)

    plsc.subcore_barrier()

    ocp = None
    for k in range(CNT_PER_TILE // ZFLAT):
        o0 = s * CNT_PER_TILE + k * ZFLAT
        if ocp is not None:
            ocp.wait()
        pltpu.sync_copy(cnt_sh.at[pl.ds(o0, ZFLAT)], zflat)
        ocp = pltpu.async_copy(
            zflat, out_c.at[pl.ds(c * CNT_WORDS + o0, ZFLAT)], p0)
    ocp.wait()


_sc_counts = functools.partial(
    pl.kernel,
    out_type=jax.ShapeDtypeStruct((NUM_CORES * CNT_WORDS,), jnp.float32),
    mesh=plsc.VectorSubcoreMesh(core_axis_name="c", subcore_axis_name="s"),
    scratch_types=[
        pltpu.VMEM((NCHUNK, CHUNK), jnp.int32),
        pltpu.VMEM((NCHUNK, CHUNK), jnp.int32),
        pltpu.VMEM((NCHUNK, CHUNK), jnp.int32),
        pltpu.VMEM((CHUNK,), jnp.float32),
        pltpu.VMEM((ZFLAT,), jnp.float32),
        pltpu.VMEM_SHARED((CNT_WORDS,), jnp.float32),
        pltpu.SemaphoreType.DMA,
        pltpu.SemaphoreType.DMA,
        pltpu.SemaphoreType.DMA,
    ],
)(_cnt_body)


BLOCK_ROWS = 1000


def _tc_body(ns_ref, s2_ref, c2_ref, rel_ref, self_w_ref, self_b_ref,
             msg_w_ref, msg_b_ref, out_ref):
    s_tot = s2_ref[0] + s2_ref[1]
    cm = c2_ref[0] + c2_ref[1]
    deg = jnp.sum(cm, axis=1, keepdims=True)
    rel_sum = lax.dot_general(cm, rel_ref[...], (((1,), (0,)), ((), ())),
                              preferred_element_type=jnp.float32)
    numer = lax.dot_general(s_tot + rel_sum, msg_w_ref[...],
                            (((1,), (1,)), ((), ())),
                            preferred_element_type=jnp.float32)
    numer = numer + deg * msg_b_ref[...]
    agg = numer / jnp.maximum(deg, 1.0)
    out_ref[...] = lax.dot_general(ns_ref[...], self_w_ref[...],
                                   (((1,), (1,)), ((), ())),
                                   preferred_element_type=jnp.float32) \
        + self_b_ref[...] + agg


def _tc_combine(ns, s2, c2, rel_emb, self_w, self_b, msg_w, msg_b):
    grid = (N_NODES // BLOCK_ROWS,)
    return pl.pallas_call(
        _tc_body,
        grid=grid,
        in_specs=[
            pl.BlockSpec((BLOCK_ROWS, HIDDEN), lambda i: (i, 0)),
            pl.BlockSpec((NUM_CORES, BLOCK_ROWS, HIDDEN), lambda i: (0, i, 0)),
            pl.BlockSpec((NUM_CORES, BLOCK_ROWS, REL_PAD), lambda i: (0, i, 0)),
            pl.BlockSpec((REL_PAD, HIDDEN), lambda i: (0, 0)),
            pl.BlockSpec((HIDDEN, HIDDEN), lambda i: (0, 0)),
            pl.BlockSpec((1, HIDDEN), lambda i: (0, 0)),
            pl.BlockSpec((HIDDEN, HIDDEN), lambda i: (0, 0)),
            pl.BlockSpec((1, HIDDEN), lambda i: (0, 0)),
        ],
        out_specs=pl.BlockSpec((BLOCK_ROWS, HIDDEN), lambda i: (i, 0)),
        out_shape=jax.ShapeDtypeStruct((N_NODES, HIDDEN), jnp.float32),
    )(ns, s2, c2, rel_emb, self_w, self_b, msg_w, msg_b)


def kernel(node_states, edge_index, edge_type_ids, self_W, self_b, msg_W, msg_b, rel_emb):
    src_flat = edge_index[0]
    dst_flat = edge_index[1]
    s2 = _sc_rows(node_states, src_flat, dst_flat)
    c2 = _sc_counts(dst_flat, edge_type_ids)
    c2 = c2.reshape(NUM_CORES, N_NODES, REL_PAD)
    return _tc_combine(node_states, s2, c2, rel_emb, self_W,
                       self_b.reshape(1, HIDDEN), msg_W, msg_b.reshape(1, HIDDEN))


# speedup vs baseline: 1.3885x; 1.0718x over previous
import functools

import jax
import jax.numpy as jnp
from jax import lax
from jax.experimental import pallas as pl
from jax.experimental.pallas import tpu as pltpu
from jax.experimental.pallas import tpu_sc as plsc

N_NODES = 10000
HIDDEN = 128
N_EDGES = 320000
NUM_REL = 64

NUM_CORES = 2
NUM_SUBCORES = 16
NW = NUM_CORES * NUM_SUBCORES
EDGES_PER_W = N_EDGES // NW
CHUNK = 80
NCHUNK = EDGES_PER_W // CHUNK
N_SBLK = N_NODES // CHUNK
REL_PAD = NUM_REL
ZFLAT = 8000
CNT_WORDS = N_NODES * REL_PAD
CNT_PER_TILE = CNT_WORDS // NUM_SUBCORES


def _rows_body(ns_hbm, ei_hbm, out_s,
               rows0, rows1, rows2, sb0, sb1, sb2, dst_all, s_sh,
               g0, g1, g2, l0, l1, l2, p0):
    c = lax.axis_index("c")
    s = lax.axis_index("s")
    w = c * NUM_SUBCORES + s
    base = w * EDGES_PER_W
    rows = (rows0, rows1, rows2)
    srcb = (sb0, sb1, sb2)
    gsem = (g0, g1, g2)
    lsem = (l0, l1, l2)

    def dfire_body(i, carry):
        pltpu.async_copy(ei_hbm.at[pl.ds(N_EDGES + base + i * CHUNK, CHUNK)],
                         dst_all.at[i], p0)
        return carry
    lax.fori_loop(0, NCHUNK, dfire_body, 0)

    def zrow_body(i, carry):
        for j in range(HIDDEN // 16):
            rows0[i, pl.ds(j * 16, 16)] = jnp.zeros((16,), jnp.float32)
        return carry
    lax.fori_loop(0, CHUNK, zrow_body, 0)

    zcps = []
    for k in range(N_SBLK // NUM_SUBCORES):
        blk = s + NUM_SUBCORES * k
        zcps.append(pltpu.async_copy(rows0, s_sh.at[pl.ds(blk * CHUNK, CHUNK)], g0))
    blk_raw = s + NUM_SUBCORES * (N_SBLK // NUM_SUBCORES)
    blk = jnp.minimum(blk_raw, N_SBLK - 1)
    @pl.when(blk_raw < N_SBLK)
    def _():
        pltpu.async_copy(rows0, s_sh.at[pl.ds(blk * CHUNK, CHUNK)], g0).wait()
    for cp in zcps:
        cp.wait()

    def ddrain_body(i, carry):
        pltpu.make_async_copy(ei_hbm.at[pl.ds(0, CHUNK)], dst_all.at[0], p0).wait()
        return carry
    lax.fori_loop(0, NCHUNK, ddrain_body, 0)

    plsc.subcore_barrier()

    def load_src(i, b):
        return pltpu.async_copy(
            ei_hbm.at[pl.ds(base + i * CHUNK, CHUNK)], srcb[b], lsem[b])

    def fire(b):
        return pltpu.async_copy(ns_hbm.at[srcb[b]], rows[b], gsem[b])

    load_src(0, 0).wait()
    fire(0)
    load_src(1, 1).wait()
    fire(1)
    load_src(2, 2)

    def tri_body(k, carry):
        for j in range(3):
            i = 3 * k + j
            b = j
            b2 = (j + 2) % 3
            pltpu.make_async_copy(ns_hbm.at[srcb[b]], rows[b], gsem[b]).wait()
            @pl.when(i + 3 < NCHUNK)
            def _():
                load_src(i + 3, b)
            pltpu.make_async_copy(
                ei_hbm.at[pl.ds(0, CHUNK)], srcb[b2], lsem[b2]).wait()
            fire(b2)
            pltpu.sync_copy(rows[b], s_sh.at[dst_all.at[i]], add=True)
        return carry
    lax.fori_loop(0, (NCHUNK - 2) // 3, tri_body, 0)

    for i in (NCHUNK - 2, NCHUNK - 1):
        b = i % 3
        pltpu.make_async_copy(ns_hbm.at[srcb[b]], rows[b], gsem[b]).wait()
        pltpu.sync_copy(rows[b], s_sh.at[dst_all.at[i]], add=True)

    plsc.subcore_barrier()

    rcps = []
    for k in range(N_SBLK // NUM_SUBCORES):
        blk = s + NUM_SUBCORES * k
        r0 = blk * CHUNK
        rcps.append(pltpu.async_copy(
            s_sh.at[pl.ds(r0, CHUNK)], out_s.at[c, pl.ds(r0, CHUNK)], p0))
    blk_raw = s + NUM_SUBCORES * (N_SBLK // NUM_SUBCORES)
    blk = jnp.minimum(blk_raw, N_SBLK - 1)
    @pl.when(blk_raw < N_SBLK)
    def _():
        r0 = blk * CHUNK
        pltpu.async_copy(
            s_sh.at[pl.ds(r0, CHUNK)], out_s.at[c, pl.ds(r0, CHUNK)], p0).wait()
    for cp in rcps:
        cp.wait()


_sc_rows = functools.partial(
    pl.kernel,
    out_type=jax.ShapeDtypeStruct((NUM_CORES, N_NODES, HIDDEN), jnp.float32),
    mesh=plsc.VectorSubcoreMesh(core_axis_name="c", subcore_axis_name="s"),
    scratch_types=[
        pltpu.VMEM((CHUNK, HIDDEN), jnp.float32),
        pltpu.VMEM((CHUNK, HIDDEN), jnp.float32),
        pltpu.VMEM((CHUNK, HIDDEN), jnp.float32),
        pltpu.VMEM((CHUNK,), jnp.int32),
        pltpu.VMEM((CHUNK,), jnp.int32),
        pltpu.VMEM((CHUNK,), jnp.int32),
        pltpu.VMEM((NCHUNK, CHUNK), jnp.int32),
        pltpu.VMEM_SHARED((N_NODES, HIDDEN), jnp.float32),
        pltpu.SemaphoreType.DMA,
        pltpu.SemaphoreType.DMA,
        pltpu.SemaphoreType.DMA,
        pltpu.SemaphoreType.DMA,
        pltpu.SemaphoreType.DMA,
        pltpu.SemaphoreType.DMA,
        pltpu.SemaphoreType.DMA,
    ],
)(_rows_body)


def _cnt_body(ei_hbm, rel_hbm, out_c,
              dst_v, rel_v, cidx_all, ones_v, zflat, cnt_sh, ssem, p0, p1):
    c = lax.axis_index("c")
    s = lax.axis_index("s")
    w = c * NUM_SUBCORES + s
    base = w * EDGES_PER_W

    for j in range(CHUNK // 16):
        ones_v[pl.ds(j * 16, 16)] = jnp.ones((16,), jnp.float32)

    def zflat_body(i, carry):
        zflat[pl.ds(i * 16, 16)] = jnp.zeros((16,), jnp.float32)
        return carry
    lax.fori_loop(0, ZFLAT // 16, zflat_body, 0)

    zcps = []
    for k in range(CNT_PER_TILE // ZFLAT):
        zcps.append(pltpu.async_copy(
            zflat, cnt_sh.at[pl.ds(s * CNT_PER_TILE + k * ZFLAT, ZFLAT)], ssem))

    def pfire_body(i, carry):
        pltpu.async_copy(ei_hbm.at[pl.ds(N_EDGES + base + i * CHUNK, CHUNK)],
                         dst_v.at[i], p0)
        pltpu.async_copy(rel_hbm.at[pl.ds(base + i * CHUNK, CHUNK)],
                         rel_v.at[i], p1)
        return carry
    lax.fori_loop(0, NCHUNK, pfire_body, 0)

    def pdrain_body(i, carry):
        pltpu.make_async_copy(ei_hbm.at[pl.ds(0, CHUNK)], dst_v.at[0], p0).wait()
        pltpu.make_async_copy(rel_hbm.at[pl.ds(0, CHUNK)], rel_v.at[0], p1).wait()
        return carry
    lax.fori_loop(0, NCHUNK, pdrain_body, 0)
    for cp in zcps:
        cp.wait()

    plsc.subcore_barrier()

    def chunk_body(i, carry):
        for u in range(CHUNK // 16):
            d16 = dst_v[i, pl.ds(u * 16, 16)]
            r16 = rel_v[i, pl.ds(u * 16, 16)]
            r16 = jnp.minimum(jnp.maximum(r16, 0), NUM_REL - 1)
            cidx_all[i, pl.ds(u * 16, 16)] = d16 * REL_PAD + r16
        pltpu.async_copy(ones_v, cnt_sh.at[cidx_all.at[i]], ssem, add=True)
        return carry
    lax.fori_loop(0, NCHUNK, chunk_body, 0)

    def drain_body(i, carry):
        pltpu.make_async_copy(ones_v, cnt_sh.at[pl.ds(0, CHUNK)], ssem).wait()
        return carry
    lax.fori_loop(0, NCHUNK, drain_body, 0)

    plsc.subcore_barrier()

    ocp = None
    for k in range(CNT_PER_TILE // ZFLAT):
        o0 = s * CNT_PER_TILE + k * ZFLAT
        if ocp is not None:
            ocp.wait()
        pltpu.sync_copy(cnt_sh.at[pl.ds(o0, ZFLAT)], zflat)
        ocp = pltpu.async_copy(
            zflat, out_c.at[pl.ds(c * CNT_WORDS + o0, ZFLAT)], p0)
    ocp.wait()


_sc_counts = functools.partial(
    pl.kernel,
    out_type=jax.ShapeDtypeStruct((NUM_CORES * CNT_WORDS,), jnp.float32),
    mesh=plsc.VectorSubcoreMesh(core_axis_name="c", subcore_axis_name="s"),
    scratch_types=[
        pltpu.VMEM((NCHUNK, CHUNK), jnp.int32),
        pltpu.VMEM((NCHUNK, CHUNK), jnp.int32),
        pltpu.VMEM((NCHUNK, CHUNK), jnp.int32),
        pltpu.VMEM((CHUNK,), jnp.float32),
        pltpu.VMEM((ZFLAT,), jnp.float32),
        pltpu.VMEM_SHARED((CNT_WORDS,), jnp.float32),
        pltpu.SemaphoreType.DMA,
        pltpu.SemaphoreType.DMA,
        pltpu.SemaphoreType.DMA,
    ],
)(_cnt_body)


BLOCK_ROWS = 1000


def _tc_body(ns_ref, s2_ref, c2_ref, rel_ref, self_w_ref, self_b_ref,
             msg_w_ref, msg_b_ref, out_ref):
    s_tot = s2_ref[0] + s2_ref[1]
    cm = c2_ref[0] + c2_ref[1]
    deg = jnp.sum(cm, axis=1, keepdims=True)
    rel_sum = lax.dot_general(cm, rel_ref[...], (((1,), (0,)), ((), ())),
                              preferred_element_type=jnp.float32)
    numer = lax.dot_general(s_tot + rel_sum, msg_w_ref[...],
                            (((1,), (1,)), ((), ())),
                            preferred_element_type=jnp.float32)
    numer = numer + deg * msg_b_ref[...]
    agg = numer / jnp.maximum(deg, 1.0)
    out_ref[...] = lax.dot_general(ns_ref[...], self_w_ref[...],
                                   (((1,), (1,)), ((), ())),
                                   preferred_element_type=jnp.float32) \
        + self_b_ref[...] + agg


def _tc_combine(ns, s2, c2, rel_emb, self_w, self_b, msg_w, msg_b):
    grid = (N_NODES // BLOCK_ROWS,)
    return pl.pallas_call(
        _tc_body,
        grid=grid,
        in_specs=[
            pl.BlockSpec((BLOCK_ROWS, HIDDEN), lambda i: (i, 0)),
            pl.BlockSpec((NUM_CORES, BLOCK_ROWS, HIDDEN), lambda i: (0, i, 0)),
            pl.BlockSpec((NUM_CORES, BLOCK_ROWS, REL_PAD), lambda i: (0, i, 0)),
            pl.BlockSpec((REL_PAD, HIDDEN), lambda i: (0, 0)),
            pl.BlockSpec((HIDDEN, HIDDEN), lambda i: (0, 0)),
            pl.BlockSpec((1, HIDDEN), lambda i: (0, 0)),
            pl.BlockSpec((HIDDEN, HIDDEN), lambda i: (0, 0)),
            pl.BlockSpec((1, HIDDEN), lambda i: (0, 0)),
        ],
        out_specs=pl.BlockSpec((BLOCK_ROWS, HIDDEN), lambda i: (i, 0)),
        out_shape=jax.ShapeDtypeStruct((N_NODES, HIDDEN), jnp.float32),
    )(ns, s2, c2, rel_emb, self_w, self_b, msg_w, msg_b)


def kernel(node_states, edge_index, edge_type_ids, self_W, self_b, msg_W, msg_b, rel_emb):
    ei_flat = edge_index.reshape(2 * N_EDGES)
    s2 = _sc_rows(node_states, ei_flat)
    c2 = _sc_counts(ei_flat, edge_type_ids)
    c2 = c2.reshape(NUM_CORES, N_NODES, REL_PAD)
    return _tc_combine(node_states, s2, c2, rel_emb, self_W,
                       self_b.reshape(1, HIDDEN), msg_W, msg_b.reshape(1, HIDDEN))
